# baseline (speedup 1.0000x reference)
import jax
import jax.numpy as jnp
from jax.experimental import pallas as pl
from jax.experimental.pallas import tpu as pltpu

B = 64
N_FRAMES = 128
FRAME_LEN = 256
T = N_FRAMES * FRAME_LEN
N_RATES = 16
RPC = 8
NCHUNK = B // RPC
NBUF = 4
KP = 16
KCOLS = KP * FRAME_LEN
NK = N_FRAMES // KP


def _copy(x_ref, bufs, sems, c):
    j = c % NBUF
    return pltpu.make_async_copy(
        x_ref.at[pl.ds(c * RPC, RPC), :], bufs.at[j], sems.at[j])


def _body(x_ref, mask_ref, rd_ref, ic_ref, w_ref, b_ref, s_ref, out_ref,
          bufs, sems):
    for c in range(NBUF):
        _copy(x_ref, bufs, sems, c).start()

    u_parts = []
    for c in range(NCHUNK):
        _copy(x_ref, bufs, sems, c).wait()
        xb = bufs[c % NBUF]
        ur = jnp.zeros((RPC, N_RATES), jnp.float32)
        for k in range(NK):
            xk = xb[:, k * KCOLS:(k + 1) * KCOLS]
            ek = jnp.dot(xk * xk, s_ref[...],
                         preferred_element_type=jnp.float32)
            emk = ek * mask_ref[pl.ds(c * RPC, RPC), pl.ds(k * KP, KP)] \
                * (1.0 / FRAME_LEN)
            ur = ur + jnp.dot(emk, w_ref[pl.ds(k * KP, KP), :],
                              preferred_element_type=jnp.float32)
        u_parts.append(ur)
        if c + NBUF < NCHUNK:
            _copy(x_ref, bufs, sems, c + NBUF).start()
    u = jnp.concatenate(u_parts, axis=0)

    rd = rd_ref[...]
    m = jnp.max(rd, axis=-1, keepdims=True)
    lane = jax.lax.broadcasted_iota(jnp.int32, rd.shape, 1)
    idx = jnp.min(jnp.where(rd == m, lane, N_RATES), axis=-1, keepdims=True)
    rate = 0.5 + 0.1 * idx.astype(jnp.float32)

    logits = rate * rate * u + b_ref[...]
    lmax = jnp.max(logits, axis=-1, keepdims=True)
    ex = jnp.exp(logits - lmax)
    sal = ex / jnp.sum(ex, axis=-1, keepdims=True)

    onehot = (lane == ic_ref[...]).astype(jnp.float32)
    sal_ic = jnp.sum(sal * onehot, axis=-1)
    l1 = 1.0 - sal_ic
    mult = m[:, 0] * jnp.log(m[:, 0])
    loss1 = jnp.sum(l1 * mult) * (1.0 / B)

    ent = jnp.sum(-rd * jnp.log(rd + 1e-12)) * (1.0 / B)
    out_ref[...] = jnp.reshape(loss1 - 0.01 * ent, (1, 1))


def kernel(x, rate_distribution, mask_sample, intent_cats, W_sal, b_sal):
    mask2 = mask_sample.reshape(B, N_FRAMES)
    ic = intent_cats.astype(jnp.int32).reshape(B, 1)
    b2 = b_sal.reshape(1, N_RATES)
    s = (jax.lax.broadcasted_iota(jnp.int32, (KCOLS, KP), 0) // FRAME_LEN
         == jax.lax.broadcasted_iota(jnp.int32, (KCOLS, KP), 1)
         ).astype(jnp.float32)

    vm = pltpu.VMEM
    out = pl.pallas_call(
        _body,
        in_specs=[
            pl.BlockSpec(memory_space=pl.ANY),
            pl.BlockSpec(memory_space=vm),
            pl.BlockSpec(memory_space=vm),
            pl.BlockSpec(memory_space=vm),
            pl.BlockSpec(memory_space=vm),
            pl.BlockSpec(memory_space=vm),
            pl.BlockSpec(memory_space=vm),
        ],
        out_specs=pl.BlockSpec(memory_space=vm),
        out_shape=jax.ShapeDtypeStruct((1, 1), jnp.float32),
        scratch_shapes=[pltpu.VMEM((NBUF, RPC, T), jnp.float32),
                        pltpu.SemaphoreType.DMA((NBUF,))],
    )(x, mask2, rate_distribution, ic, W_sal, b2, s)
    return out[0, 0]


# speedup vs baseline: 1.4624x; 1.4624x over previous
import jax
import jax.numpy as jnp
from jax.experimental import pallas as pl
from jax.experimental.pallas import tpu as pltpu

B = 64
N_FRAMES = 128
FRAME_LEN = 256
T = N_FRAMES * FRAME_LEN
N_RATES = 16
FPB = 16
COLS = FPB * FRAME_LEN
NCHUNK = N_FRAMES // FPB
NBUF = 8


def _copy(x_ref, bufs, sems, c):
    j = c % NBUF
    return pltpu.make_async_copy(
        x_ref.at[:, pl.ds(c * COLS, COLS)], bufs.at[j], sems.at[j])


def _body(x_ref, mask_ref, rd_ref, ic_ref, w_ref, b_ref, s_ref, out_ref,
          bufs, sems):
    for c in range(NBUF):
        _copy(x_ref, bufs, sems, c).start()

    mask2 = mask_ref[...][:, :, 0]
    u = jnp.zeros((B, N_RATES), jnp.float32)
    for c in range(NCHUNK):
        _copy(x_ref, bufs, sems, c).wait()
        xb = bufs[c % NBUF]
        eb = jnp.dot(xb * xb, s_ref[...],
                     preferred_element_type=jnp.float32)
        em = eb * mask2[:, c * FPB:(c + 1) * FPB] * (1.0 / FRAME_LEN)
        u = u + jnp.dot(em, w_ref[pl.ds(c * FPB, FPB), :],
                        preferred_element_type=jnp.float32)
        if c + NBUF < NCHUNK:
            _copy(x_ref, bufs, sems, c + NBUF).start()

    rd = rd_ref[...]
    m = jnp.max(rd, axis=-1, keepdims=True)
    lane = jax.lax.broadcasted_iota(jnp.int32, rd.shape, 1)
    idx = jnp.min(jnp.where(rd == m, lane, N_RATES), axis=-1, keepdims=True)
    rate = 0.5 + 0.1 * idx.astype(jnp.float32)

    logits = rate * rate * u + b_ref[...][None, :]
    lmax = jnp.max(logits, axis=-1, keepdims=True)
    ex = jnp.exp(logits - lmax)
    sal = ex / jnp.sum(ex, axis=-1, keepdims=True)

    onehot = (lane == ic_ref[...][:, None]).astype(jnp.float32)
    sal_ic = jnp.sum(sal * onehot, axis=-1)
    l1 = 1.0 - sal_ic
    mult = m[:, 0] * jnp.log(m[:, 0])
    loss1 = jnp.sum(l1 * mult) * (1.0 / B)

    ent = jnp.sum(-rd * jnp.log(rd + 1e-12)) * (1.0 / B)
    out_ref[...] = jnp.reshape(loss1 - 0.01 * ent, (1, 1))


def kernel(x, rate_distribution, mask_sample, intent_cats, W_sal, b_sal):
    s = (jax.lax.broadcasted_iota(jnp.int32, (COLS, FPB), 0) // FRAME_LEN
         == jax.lax.broadcasted_iota(jnp.int32, (COLS, FPB), 1)
         ).astype(jnp.float32)

    vm = pltpu.VMEM
    out = pl.pallas_call(
        _body,
        in_specs=[
            pl.BlockSpec(memory_space=pl.ANY),
            pl.BlockSpec(memory_space=vm),
            pl.BlockSpec(memory_space=vm),
            pl.BlockSpec(memory_space=vm),
            pl.BlockSpec(memory_space=vm),
            pl.BlockSpec(memory_space=vm),
            pl.BlockSpec(memory_space=vm),
        ],
        out_specs=pl.BlockSpec(memory_space=vm),
        out_shape=jax.ShapeDtypeStruct((1, 1), jnp.float32),
        scratch_shapes=[pltpu.VMEM((NBUF, B, COLS), jnp.float32),
                        pltpu.SemaphoreType.DMA((NBUF,))],
    )(x, mask_sample, rate_distribution, intent_cats.astype(jnp.int32),
      W_sal, b_sal, s)
    return out[0, 0]


# speedup vs baseline: 1.7949x; 1.2273x over previous
import jax
import jax.numpy as jnp
from jax.experimental import pallas as pl
from jax.experimental.pallas import tpu as pltpu

B = 64
N_FRAMES = 128
FRAME_LEN = 256
T = N_FRAMES * FRAME_LEN
N_RATES = 16
FPB = 16
COLS = FPB * FRAME_LEN
NCHUNK = N_FRAMES // FPB
NBUF = 8


HB = B // 2


def _copies(x_ref, bufs, sems, c):
    j = c % NBUF
    cols = pl.ds(c * COLS, COLS)
    return (
        pltpu.make_async_copy(x_ref.at[pl.ds(0, HB), cols],
                              bufs.at[j, pl.ds(0, HB)], sems.at[j, 0]),
        pltpu.make_async_copy(x_ref.at[pl.ds(HB, HB), cols],
                              bufs.at[j, pl.ds(HB, HB)], sems.at[j, 1]),
    )


def _body(x_ref, mask_ref, rd_ref, ic_ref, w_ref, b_ref, s_ref, out_ref,
          bufs, sems):
    for c in range(NBUF):
        for cp in _copies(x_ref, bufs, sems, c):
            cp.start()

    u = jnp.zeros((B, N_RATES), jnp.float32)
    for c in range(NCHUNK):
        for cp in _copies(x_ref, bufs, sems, c):
            cp.wait()
        xb = bufs[c % NBUF]
        eb = jnp.dot(xb * xb, s_ref[...],
                     preferred_element_type=jnp.float32)
        em = eb * mask_ref[c] * (1.0 / FRAME_LEN)
        u = u + jnp.dot(em, w_ref[pl.ds(c * FPB, FPB), :],
                        preferred_element_type=jnp.float32)
        if c + NBUF < NCHUNK:
            for cp in _copies(x_ref, bufs, sems, c + NBUF):
                cp.start()

    rd = rd_ref[...]
    m = jnp.max(rd, axis=-1, keepdims=True)
    lane = jax.lax.broadcasted_iota(jnp.int32, rd.shape, 1)
    idx = jnp.min(jnp.where(rd == m, lane, N_RATES), axis=-1, keepdims=True)
    rate = 0.5 + 0.1 * idx.astype(jnp.float32)

    logits = rate * rate * u + b_ref[...]
    lmax = jnp.max(logits, axis=-1, keepdims=True)
    ex = jnp.exp(logits - lmax)
    sal = ex / jnp.sum(ex, axis=-1, keepdims=True)

    onehot = (lane == ic_ref[...]).astype(jnp.float32)
    sal_ic = jnp.sum(sal * onehot, axis=-1)
    l1 = 1.0 - sal_ic
    mult = m[:, 0] * jnp.log(m[:, 0])
    loss1 = jnp.sum(l1 * mult) * (1.0 / B)

    ent = jnp.sum(-rd * jnp.log(rd + 1e-12)) * (1.0 / B)
    out_ref[...] = jnp.reshape(loss1 - 0.01 * ent, (1, 1))


def kernel(x, rate_distribution, mask_sample, intent_cats, W_sal, b_sal):
    mask3 = mask_sample.reshape(B, NCHUNK, FPB).transpose(1, 0, 2)
    ic = intent_cats.astype(jnp.int32).reshape(B, 1)
    b2 = b_sal.reshape(1, N_RATES)
    s = (jax.lax.broadcasted_iota(jnp.int32, (COLS, FPB), 0) // FRAME_LEN
         == jax.lax.broadcasted_iota(jnp.int32, (COLS, FPB), 1)
         ).astype(jnp.float32)

    vm = pltpu.VMEM
    out = pl.pallas_call(
        _body,
        in_specs=[
            pl.BlockSpec(memory_space=pl.ANY),
            pl.BlockSpec(memory_space=vm),
            pl.BlockSpec(memory_space=vm),
            pl.BlockSpec(memory_space=vm),
            pl.BlockSpec(memory_space=vm),
            pl.BlockSpec(memory_space=vm),
            pl.BlockSpec(memory_space=vm),
        ],
        out_specs=pl.BlockSpec(memory_space=vm),
        out_shape=jax.ShapeDtypeStruct((1, 1), jnp.float32),
        scratch_shapes=[pltpu.VMEM((NBUF, B, COLS), jnp.float32),
                        pltpu.SemaphoreType.DMA((NBUF, 2))],
    )(x, mask3, rate_distribution, ic, W_sal, b2, s)
    return out[0, 0]
